# baseline (speedup 1.0000x reference)
import jax
import jax.numpy as jnp
from jax.experimental import pallas as pl
from jax.experimental.pallas import tpu as pltpu

_M_BLK = 200


def _proj_kernel(x_ref, w_ref, o_ref):
    o_ref[...] = jnp.dot(x_ref[...], w_ref[...], preferred_element_type=jnp.float32)


def _gcn1_kernel(adj_ref, s_ref, b_ref, w2_ref, h_ref, t_ref):
    acc = jnp.dot(adj_ref[...], s_ref[...], preferred_element_type=jnp.float32)
    h = jnp.maximum(acc + b_ref[...], 0.0)
    h_ref[...] = h
    t_ref[...] = jnp.dot(h, w2_ref[...], preferred_element_type=jnp.float32)


def _gcn2_kernel(adj_ref, t_ref, b_ref, wp_ref, x_ref, p_ref):
    acc = jnp.dot(adj_ref[...], t_ref[...], preferred_element_type=jnp.float32)
    acc = acc + b_ref[...]
    x_ref[...] = acc
    r = jnp.maximum(acc, 0.0)
    p_ref[...] = jnp.maximum(
        jnp.dot(r, wp_ref[...], preferred_element_type=jnp.float32), 0.0
    )


def _proj(x, w):
    n, f = x.shape
    return pl.pallas_call(
        _proj_kernel,
        out_shape=jax.ShapeDtypeStruct((n, w.shape[1]), jnp.float32),
    )(x, w)


def _gcn_branch(adj, s, b1, w2, b2, wp):
    n = adj.shape[0]
    grid = (n // _M_BLK,)
    nh1 = s.shape[1]
    nh2 = w2.shape[1]
    ncls = wp.shape[1]

    h, t = pl.pallas_call(
        _gcn1_kernel,
        grid=grid,
        in_specs=[
            pl.BlockSpec((_M_BLK, n), lambda i: (i, 0)),
            pl.BlockSpec((n, nh1), lambda i: (0, 0)),
            pl.BlockSpec((1, nh1), lambda i: (0, 0)),
            pl.BlockSpec((nh1, nh2), lambda i: (0, 0)),
        ],
        out_specs=[
            pl.BlockSpec((_M_BLK, nh1), lambda i: (i, 0)),
            pl.BlockSpec((_M_BLK, nh2), lambda i: (i, 0)),
        ],
        out_shape=[
            jax.ShapeDtypeStruct((n, nh1), jnp.float32),
            jax.ShapeDtypeStruct((n, nh2), jnp.float32),
        ],
        compiler_params=pltpu.CompilerParams(
            dimension_semantics=("arbitrary",),
        ),
    )(adj, s, b1.reshape(1, -1), w2)

    x, p = pl.pallas_call(
        _gcn2_kernel,
        grid=grid,
        in_specs=[
            pl.BlockSpec((_M_BLK, n), lambda i: (i, 0)),
            pl.BlockSpec((n, nh2), lambda i: (0, 0)),
            pl.BlockSpec((1, nh2), lambda i: (0, 0)),
            pl.BlockSpec((nh2, ncls), lambda i: (0, 0)),
        ],
        out_specs=[
            pl.BlockSpec((_M_BLK, nh2), lambda i: (i, 0)),
            pl.BlockSpec((_M_BLK, ncls), lambda i: (i, 0)),
        ],
        out_shape=[
            jax.ShapeDtypeStruct((n, nh2), jnp.float32),
            jax.ShapeDtypeStruct((n, ncls), jnp.float32),
        ],
        compiler_params=pltpu.CompilerParams(
            dimension_semantics=("arbitrary",),
        ),
    )(adj, t, b2.reshape(1, -1), wp)
    return x, p


def kernel(X, nsadj, nfadj, W1a, b1a, W2a, b2a, W1b, b1b, W2b, b2b, Wp):
    sa = _proj(X, W1a)
    sb = _proj(X, W1b)
    x1, p1 = _gcn_branch(nsadj, sa, b1a, W2a, b2a, Wp)
    x2, p2 = _gcn_branch(nfadj, sb, b1b, W2b, b2b, Wp)
    return (p1, p2, x1, x2)
